# baseline (speedup 1.0000x reference)
import functools

import jax
import jax.numpy as jnp
from jax import lax
from jax.experimental import pallas as pl
from jax.experimental.pallas import tpu as pltpu
from jax.experimental.pallas import tpu_sc as plsc

_K = 9
_D = 2


def _make_sc_kernel(two, b, n, dtype):
    mesh = plsc.VectorSubcoreMesh(core_axis_name="c", subcore_axis_name="s")

    @functools.partial(
        pl.kernel,
        mesh=mesh,
        out_type=jax.ShapeDtypeStruct((two, _K, b, n), dtype),
        scratch_types=[
            pltpu.VMEM((1, 1, b, n), jnp.int32),
            pltpu.SemaphoreType.DMA,
        ],
    )
    def sc_copy(t_hbm, out_hbm, buf, sem):
        c = lax.axis_index("c")
        s = lax.axis_index("s")
        wid = s * 2 + c

        @pl.when(wid < two * _K)
        def _():
            d0 = wid // _K
            k = wid % _K
            pltpu.async_copy(
                t_hbm.at[pl.ds(d0, 1), pl.ds(_D * k, 1), :, :], buf, sem
            ).wait()
            pltpu.async_copy(
                buf, out_hbm.at[pl.ds(d0, 1), pl.ds(k, 1), :, :], sem
            ).wait()

    return sc_copy


def kernel(edge_index):
    two, b, n, kd = edge_index.shape
    t = jnp.transpose(edge_index, (0, 3, 1, 2))
    out_t = _make_sc_kernel(two, b, n, edge_index.dtype)(t)
    return jnp.transpose(out_t, (0, 2, 3, 1))


# speedup vs baseline: 1.0256x; 1.0256x over previous
import functools

import jax
import jax.numpy as jnp
from jax import lax
from jax.experimental import pallas as pl
from jax.experimental.pallas import tpu as pltpu
from jax.experimental.pallas import tpu_sc as plsc

_K = 9
_D = 2
_NW = 32
_CHUNKS_PER_SLAB = 8


def _make_sc_kernel(two, b, n, dtype):
    mesh = plsc.VectorSubcoreMesh(core_axis_name="c", subcore_axis_name="s")
    nchunks = two * _K * _CHUNKS_PER_SLAB
    per_w = nchunks // _NW
    rem = nchunks % _NW

    @functools.partial(
        pl.kernel,
        mesh=mesh,
        out_type=jax.ShapeDtypeStruct((two, _K, b, n), dtype),
        scratch_types=[
            pltpu.VMEM((2, 1, 1, 1, n), jnp.int32),
            pltpu.SemaphoreType.DMA((2,)),
            pltpu.SemaphoreType.DMA((2,)),
        ],
    )
    def sc_copy(t_hbm, out_hbm, buf, in_sems, out_sems):
        c = lax.axis_index("c")
        s = lax.axis_index("s")
        wid = s * 2 + c
        base = wid * per_w + jnp.minimum(wid, rem)
        count = per_w + jnp.where(wid < rem, 1, 0)

        def chunk_refs(i):
            d0 = i // (_K * _CHUNKS_PER_SLAB)
            r = i % (_K * _CHUNKS_PER_SLAB)
            k = r // _CHUNKS_PER_SLAB
            row = r % _CHUNKS_PER_SLAB
            src = t_hbm.at[pl.ds(d0, 1), pl.ds(_D * k, 1), pl.ds(row, 1), :]
            dst = out_hbm.at[pl.ds(d0, 1), pl.ds(k, 1), pl.ds(row, 1), :]
            return src, dst

        def start_in(j, slot):
            src, _ = chunk_refs(base + j)
            pltpu.make_async_copy(
                src, buf.at[slot], in_sems.at[slot]
            ).start()

        def wait_in(j, slot):
            src, _ = chunk_refs(base + j)
            pltpu.make_async_copy(
                src, buf.at[slot], in_sems.at[slot]
            ).wait()

        def start_out(j, slot):
            _, dst = chunk_refs(base + j)
            pltpu.make_async_copy(
                buf.at[slot], dst, out_sems.at[slot]
            ).start()

        def wait_out(j, slot):
            _, dst = chunk_refs(base + j)
            pltpu.make_async_copy(
                buf.at[slot], dst, out_sems.at[slot]
            ).wait()

        @pl.when(count > 0)
        def _():
            start_in(0, 0)

            def body(j, _):
                slot = lax.rem(j, 2)
                nslot = lax.rem(j + 1, 2)

                @pl.when(j >= 1)
                def _():
                    wait_out(j - 1, nslot)

                @pl.when(j + 1 < count)
                def _():
                    start_in(j + 1, nslot)

                wait_in(j, slot)
                start_out(j, slot)
                return 0

            lax.fori_loop(0, count, body, 0)

            wait_out(count - 1, lax.rem(count - 1, 2))

    return sc_copy


def kernel(edge_index):
    two, b, n, kd = edge_index.shape
    t = jnp.transpose(edge_index, (0, 3, 1, 2))
    out_t = _make_sc_kernel(two, b, n, edge_index.dtype)(t)
    return jnp.transpose(out_t, (0, 2, 3, 1))


# speedup vs baseline: 5.5460x; 5.4075x over previous
import jax
import jax.numpy as jnp
from jax.experimental import pallas as pl
from jax.experimental.pallas import tpu as pltpu

_K = 9
_D = 2


def _copy_kernel(*refs):
    out_ref = refs[_K]
    for k in range(_K):
        out_ref[:, k : k + 1, :, :] = refs[k][...]


def _slab_spec(k, b, n):
    return pl.BlockSpec((1, 1, b, n), lambda i, _k=k: (i, _D * _k, 0, 0))


def kernel(edge_index):
    two, b, n, kd = edge_index.shape
    t = jnp.transpose(edge_index, (0, 3, 1, 2))
    out_t = pl.pallas_call(
        _copy_kernel,
        grid=(two,),
        in_specs=[_slab_spec(k, b, n) for k in range(_K)],
        out_specs=pl.BlockSpec((1, _K, b, n), lambda i: (i, 0, 0, 0)),
        out_shape=jax.ShapeDtypeStruct((two, _K, b, n), edge_index.dtype),
    )(*([t] * _K))
    return jnp.transpose(out_t, (0, 2, 3, 1))
